# baseline (speedup 1.0000x reference)
import functools

import jax
import jax.numpy as jnp
from jax import lax
from jax.experimental import pallas as pl
from jax.experimental.pallas import tpu as pltpu
from jax.experimental.pallas import tpu_sc as plsc

N_NODES = 10000
N_EDGES = 320000
D_FEAT = 128

NC = 2
NS = 16
NW = NC * NS

CHUNK = 128
EPW = N_EDGES // NW
NCH = 80
EPW_PAD = NCH * CHUNK
PAD = EPW_PAD - EPW
BLK_CH = 8
NBLK = NCH // BLK_CH
BLK_E = BLK_CH * CHUNK
N_DUMP = 8
N_ACC = N_NODES + N_DUMP
N_CNT = N_ACC + 8
NPT = 640


def _sc_body(feat_hbm, srch_hbm, dsth_hbm, z_hbm,
             part_hbm, cnt_hbm,
             sblk0, sblk1, dblk0, dblk1, rows0, rows1,
             src_c0, src_c1, dst_c0, dst_c1, cnt_v, acc_sh,
             gsem0, gsem1, isem0, isem1):
    c = lax.axis_index("c")
    s = lax.axis_index("s")
    wid = c * NS + s

    nb = pl.multiple_of(
        jnp.minimum(s * NPT, N_NODES - NPT).astype(jnp.int32), 8)
    pltpu.sync_copy(z_hbm, acc_sh.at[pl.ds(nb, NPT)])

    zero16 = jnp.zeros((16,), jnp.float32)

    def zstep(i, _):
        cnt_v[pl.ds(i * 16, 16)] = zero16
        return 0

    lax.fori_loop(0, N_CNT // 16, zstep, 0)
    plsc.subcore_barrier()

    ones16 = jnp.ones((16,), jnp.float32)

    idx_bufs = ((sblk0, dblk0, isem0), (sblk1, dblk1, isem1))
    row_bufs = ((rows0, gsem0, src_c0, dst_c0), (rows1, gsem1, src_c1, dst_c1))

    def reg_copy(blk, j, root):
        for v in range(CHUNK // 16):
            root[pl.ds(v * 16, 16)] = blk[j, 0, pl.ds(v * 16, 16)]

    def issue_idx(kb, ib):
        sb, db, isem = ib
        pltpu.async_copy(srch_hbm.at[wid, kb], sb, isem)
        pltpu.async_copy(dsth_hbm.at[wid, kb], db, isem)

    def wait_idx(kb, ib):
        sb, db, isem = ib
        pltpu.make_async_copy(srch_hbm.at[wid, kb], sb, isem).wait()
        pltpu.make_async_copy(dsth_hbm.at[wid, kb], db, isem).wait()

    def issue_gather(rb):
        rows_v, gsem, src_c, _ = rb
        pltpu.async_copy(feat_hbm.at[src_c], rows_v, gsem)

    def wait_gather(rb):
        rows_v, gsem, src_c, _ = rb
        pltpu.make_async_copy(feat_hbm.at[src_c], rows_v, gsem).wait()

    issue_idx(0, idx_bufs[0])
    wait_idx(0, idx_bufs[0])
    reg_copy(sblk0, 0, src_c0)
    issue_gather(row_bufs[0])

    def do_chunk(k, rb, nrb):
        j = k % BLK_CH
        kb = k // BLK_CH
        bp = kb % 2
        rows_v, _, _, dst_c = rb
        _, _, nsrc_c, _ = nrb
        wait_gather(rb)

        @pl.when(jnp.logical_and(j == 0, k + BLK_CH < NCH))
        def _():
            @pl.when(bp == 0)
            def _():
                issue_idx(kb + 1, idx_bufs[1])

            @pl.when(bp == 1)
            def _():
                issue_idx(kb + 1, idx_bufs[0])

        @pl.when(k + 1 < NCH)
        def _():
            @pl.when(jnp.logical_and(j + 1 < BLK_CH, bp == 0))
            def _():
                reg_copy(sblk0, j + 1, nsrc_c)

            @pl.when(jnp.logical_and(j + 1 < BLK_CH, bp == 1))
            def _():
                reg_copy(sblk1, j + 1, nsrc_c)

            @pl.when(jnp.logical_and(j + 1 >= BLK_CH, bp == 0))
            def _():
                wait_idx(kb + 1, idx_bufs[1])
                reg_copy(sblk1, 0, nsrc_c)

            @pl.when(jnp.logical_and(j + 1 >= BLK_CH, bp == 1))
            def _():
                wait_idx(kb + 1, idx_bufs[0])
                reg_copy(sblk0, 0, nsrc_c)

            issue_gather(nrb)

        @pl.when(bp == 0)
        def _():
            reg_copy(dblk0, j, dst_c)

        @pl.when(bp == 1)
        def _():
            reg_copy(dblk1, j, dst_c)

        pltpu.sync_copy(rows_v, acc_sh.at[dst_c], add=True)
        for v in range(CHUNK // 16):
            dvec = dst_c[pl.ds(v * 16, 16)]
            plsc.addupdate_scatter(cnt_v, [dvec], ones16)

    def estep(k, _):
        cp = k % 2

        @pl.when(cp == 0)
        def _():
            do_chunk(k, row_bufs[0], row_bufs[1])

        @pl.when(cp == 1)
        def _():
            do_chunk(k, row_bufs[1], row_bufs[0])

        return 0

    lax.fori_loop(0, NCH, estep, 0)
    plsc.subcore_barrier()

    pltpu.sync_copy(acc_sh.at[pl.ds(nb, NPT)], part_hbm.at[c, pl.ds(nb, NPT)])
    cb = pl.multiple_of(wid * N_NODES, 8)
    pltpu.sync_copy(cnt_v.at[pl.ds(0, N_NODES)],
                    cnt_hbm.at[pl.ds(cb, N_NODES)])


_sc_aggregate = functools.partial(
    pl.kernel,
    out_type=(
        jax.ShapeDtypeStruct((NC, N_NODES, D_FEAT), jnp.float32),
        jax.ShapeDtypeStruct((NW * N_NODES,), jnp.float32),
    ),
    mesh=plsc.VectorSubcoreMesh(core_axis_name="c", subcore_axis_name="s"),
    compiler_params=pltpu.CompilerParams(needs_layout_passes=False),
    scratch_types=[
        pltpu.VMEM((BLK_CH, 1, CHUNK), jnp.int32),
        pltpu.VMEM((BLK_CH, 1, CHUNK), jnp.int32),
        pltpu.VMEM((BLK_CH, 1, CHUNK), jnp.int32),
        pltpu.VMEM((BLK_CH, 1, CHUNK), jnp.int32),
        pltpu.VMEM((CHUNK, D_FEAT), jnp.float32),
        pltpu.VMEM((CHUNK, D_FEAT), jnp.float32),
        pltpu.VMEM((CHUNK,), jnp.int32),
        pltpu.VMEM((CHUNK,), jnp.int32),
        pltpu.VMEM((CHUNK,), jnp.int32),
        pltpu.VMEM((CHUNK,), jnp.int32),
        pltpu.VMEM((N_CNT,), jnp.float32),
        pltpu.VMEM_SHARED((N_ACC, D_FEAT), jnp.float32),
        pltpu.SemaphoreType.DMA,
        pltpu.SemaphoreType.DMA,
        pltpu.SemaphoreType.DMA,
        pltpu.SemaphoreType.DMA,
    ],
)(_sc_body)


def _combine_body(p0_ref, p1_ref, cnt_ref, o_ref):
    cnt = jnp.sum(cnt_ref[...], axis=0)
    total = p0_ref[...] + p1_ref[...]
    o_ref[...] = total / jnp.maximum(cnt, 1.0)[:, None]


_combine = pl.pallas_call(
    _combine_body,
    out_shape=jax.ShapeDtypeStruct((N_NODES, D_FEAT), jnp.float32),
)


@jax.jit
def kernel(feature, edge_index):
    srcw = edge_index[0].reshape(NW, EPW)
    dstw = edge_index[1].reshape(NW, EPW)
    pad_s = jnp.zeros((NW, PAD), jnp.int32)
    pad_d = jnp.broadcast_to(
        N_NODES + (jnp.arange(PAD, dtype=jnp.int32) % N_DUMP), (NW, PAD))
    shape5 = (NW, NBLK, BLK_CH, 1, CHUNK)
    src_p = jnp.concatenate([srcw, pad_s], axis=1).reshape(shape5)
    dst_p = jnp.concatenate([dstw, pad_d], axis=1).reshape(shape5)
    z = jnp.zeros((NPT, D_FEAT), jnp.float32)
    partial, cnt = _sc_aggregate(feature, src_p, dst_p, z)
    return _combine(partial[0], partial[1], cnt.reshape(NW, N_NODES))


# speedup vs baseline: 2.8090x; 2.8090x over previous
import functools

import jax
import jax.numpy as jnp
from jax import lax
from jax.experimental import pallas as pl
from jax.experimental.pallas import tpu as pltpu
from jax.experimental.pallas import tpu_sc as plsc

N_NODES = 10000
N_EDGES = 320000
D_FEAT = 128

NC = 2
NS = 16
NW = NC * NS

CHUNK = 128
EPW = N_EDGES // NW
NCHUNK = EPW // CHUNK
REM = EPW - NCHUNK * CHUNK
NPT = 640


def _sc_body(feat_hbm, edge_hbm, z_hbm,
             part_hbm, cnt_hbm,
             src0, src1, src2, src3, dst0, dst1, dst2, dst3,
             rows0, rows1, src_r, dst_r, rows_r, cnt_v, acc_sh,
             gsem0, gsem1, isem0, isem1, isem2, isem3):
    c = lax.axis_index("c")
    s = lax.axis_index("s")
    wid = c * NS + s
    ebase = wid * EPW

    ibufs = ((src0, dst0, isem0), (src1, dst1, isem1),
             (src2, dst2, isem2), (src3, dst3, isem3))
    rbufs = ((rows0, gsem0), (rows1, gsem1))

    def issue_idx(k, ib):
        src_v, dst_v, isem = ib
        b = pl.multiple_of(ebase + k * CHUNK, 8)
        b2 = pl.multiple_of(N_EDGES + ebase + k * CHUNK, 8)
        pltpu.async_copy(edge_hbm.at[pl.ds(b, CHUNK)], src_v, isem)
        pltpu.async_copy(edge_hbm.at[pl.ds(b2, CHUNK)], dst_v, isem)

    def wait_idx(k, ib):
        src_v, dst_v, isem = ib
        b = pl.multiple_of(ebase + k * CHUNK, 8)
        b2 = pl.multiple_of(N_EDGES + ebase + k * CHUNK, 8)
        pltpu.make_async_copy(edge_hbm.at[pl.ds(b, CHUNK)], src_v, isem).wait()
        pltpu.make_async_copy(edge_hbm.at[pl.ds(b2, CHUNK)], dst_v, isem).wait()

    def issue_gather(ib, rb):
        src_v, _, _ = ib
        rows_v, gsem = rb
        pltpu.async_copy(feat_hbm.at[src_v], rows_v, gsem)

    def wait_gather(ib, rb):
        src_v, _, _ = ib
        rows_v, gsem = rb
        pltpu.make_async_copy(feat_hbm.at[src_v], rows_v, gsem).wait()

    for q in range(4):
        issue_idx(q, ibufs[q])

    nb = pl.multiple_of(
        jnp.minimum(s * NPT, N_NODES - NPT).astype(jnp.int32), 8)
    pltpu.sync_copy(z_hbm, acc_sh.at[pl.ds(nb, NPT)])

    zero16 = jnp.zeros((16,), jnp.float32)

    def zstep(i, _):
        cnt_v[pl.ds(i * 16, 16)] = zero16
        return 0

    lax.fori_loop(0, N_NODES // 16, zstep, 0)

    wait_idx(0, ibufs[0])
    issue_gather(ibufs[0], rbufs[0])
    plsc.subcore_barrier()

    ones16 = jnp.ones((16,), jnp.float32)

    def do_chunk(k, ib, nib, rb, nrb):
        _, dst_c, _ = ib
        rows_c, _ = rb
        wait_gather(ib, rb)

        @pl.when(k + 1 < NCHUNK)
        def _():
            wait_idx(k + 1, nib)
            issue_gather(nib, nrb)

        pltpu.sync_copy(rows_c, acc_sh.at[dst_c], add=True)
        for v in range(CHUNK // 16):
            dvec = dst_c[pl.ds(v * 16, 16)]
            plsc.addupdate_scatter(cnt_v, [dvec], ones16)

        @pl.when(k + 4 < NCHUNK)
        def _():
            issue_idx(k + 4, ib)

    def estep(k, _):
        for q in range(4):
            @pl.when(k % 4 == q)
            def _(q=q):
                do_chunk(k, ibufs[q], ibufs[(q + 1) % 4],
                         rbufs[q % 2], rbufs[(q + 1) % 2])

        return 0

    lax.fori_loop(0, NCHUNK, estep, 0)

    rb = pl.multiple_of(ebase + NCHUNK * CHUNK, 8)
    rb2 = pl.multiple_of(N_EDGES + ebase + NCHUNK * CHUNK, 8)
    pltpu.sync_copy(edge_hbm.at[pl.ds(rb, REM)], src_r)
    pltpu.sync_copy(edge_hbm.at[pl.ds(rb2, REM)], dst_r)
    pltpu.async_copy(feat_hbm.at[src_r], rows_r, gsem0).wait()
    pltpu.sync_copy(rows_r, acc_sh.at[dst_r], add=True)
    for v in range(REM // 16):
        plsc.addupdate_scatter(cnt_v, [dst_r[pl.ds(v * 16, 16)]], ones16)

    plsc.subcore_barrier()

    pltpu.sync_copy(acc_sh.at[pl.ds(nb, NPT)], part_hbm.at[c, pl.ds(nb, NPT)])
    cb = pl.multiple_of(wid * N_NODES, 8)
    pltpu.sync_copy(cnt_v, cnt_hbm.at[pl.ds(cb, N_NODES)])


_sc_aggregate = functools.partial(
    pl.kernel,
    out_type=(
        jax.ShapeDtypeStruct((NC, N_NODES, D_FEAT), jnp.float32),
        jax.ShapeDtypeStruct((NW * N_NODES,), jnp.float32),
    ),
    mesh=plsc.VectorSubcoreMesh(core_axis_name="c", subcore_axis_name="s"),
    compiler_params=pltpu.CompilerParams(needs_layout_passes=False),
    scratch_types=[
        pltpu.VMEM((CHUNK,), jnp.int32),
        pltpu.VMEM((CHUNK,), jnp.int32),
        pltpu.VMEM((CHUNK,), jnp.int32),
        pltpu.VMEM((CHUNK,), jnp.int32),
        pltpu.VMEM((CHUNK,), jnp.int32),
        pltpu.VMEM((CHUNK,), jnp.int32),
        pltpu.VMEM((CHUNK,), jnp.int32),
        pltpu.VMEM((CHUNK,), jnp.int32),
        pltpu.VMEM((CHUNK, D_FEAT), jnp.float32),
        pltpu.VMEM((CHUNK, D_FEAT), jnp.float32),
        pltpu.VMEM((REM,), jnp.int32),
        pltpu.VMEM((REM,), jnp.int32),
        pltpu.VMEM((REM, D_FEAT), jnp.float32),
        pltpu.VMEM((N_NODES,), jnp.float32),
        pltpu.VMEM_SHARED((N_NODES, D_FEAT), jnp.float32),
        pltpu.SemaphoreType.DMA,
        pltpu.SemaphoreType.DMA,
        pltpu.SemaphoreType.DMA,
        pltpu.SemaphoreType.DMA,
        pltpu.SemaphoreType.DMA,
        pltpu.SemaphoreType.DMA,
    ],
)(_sc_body)


def _combine_body(p0_ref, p1_ref, cnt_ref, o_ref):
    cnt = jnp.sum(cnt_ref[...], axis=0)
    total = p0_ref[...] + p1_ref[...]
    o_ref[...] = total / jnp.maximum(cnt, 1.0)[:, None]


_combine = pl.pallas_call(
    _combine_body,
    out_shape=jax.ShapeDtypeStruct((N_NODES, D_FEAT), jnp.float32),
)


@jax.jit
def kernel(feature, edge_index):
    edges = edge_index.reshape(2 * N_EDGES)
    z = jnp.zeros((NPT, D_FEAT), jnp.float32)
    partial, cnt = _sc_aggregate(feature, edges, z)
    return _combine(partial[0], partial[1], cnt.reshape(NW, N_NODES))
